# baseline (speedup 1.0000x reference)
import jax
import jax.numpy as jnp
import numpy as np
from jax import lax
from jax.experimental import pallas as pl
from jax.experimental.pallas import tpu as pltpu
from jax.experimental.pallas import tpu_sc as plsc

_B, _K, _F, _NN = 8, 64, 1024, 3
_SIGMA = 0.2
_NEG_INV = -1.0 / (2.0 * _SIGMA * _SIGMA)
_NWORKERS = 16
_CHUNKS_PER_B = _NWORKERS // _B
_CHUNK = _F // _CHUNKS_PER_B


def _sc_gather_body(normals_hbm, idx_hbm, out_hbm, tbl_v, idx_v, out_v, sem):
    wid = lax.axis_index("s")
    b = wid // _CHUNKS_PER_B
    q = wid % _CHUNKS_PER_B
    cps = [
        pltpu.async_copy(normals_hbm.at[pl.ds(b * 3 * _F, 3 * _F)], tbl_v, sem),
        pltpu.async_copy(
            idx_hbm.at[pl.ds((b * _CHUNKS_PER_B + q) * _NN * _CHUNK,
                             _NN * _CHUNK)], idx_v, sem),
    ]
    for cp in cps:
        cp.wait()
    for c in range(3):
        for j in range(_NN):
            for i in range(_CHUNK // 16):
                iv = idx_v[pl.ds(j * _CHUNK + i * 16, 16)]
                out_v[pl.ds((c * _NN + j) * _CHUNK + i * 16, 16)] = (
                    plsc.load_gather(tbl_v, [iv + c * _F]))
    pltpu.sync_copy(
        out_v,
        out_hbm.at[pl.ds((b * _CHUNKS_PER_B + q) * 9 * _CHUNK, 9 * _CHUNK)])


_sc_gather_cache = []


def _sc_gather(normals_flat, idx_flat):
    if not _sc_gather_cache:
        _sc_gather_cache.append(pl.kernel(
            _sc_gather_body,
            mesh=plsc.VectorSubcoreMesh(core_axis_name="c", subcore_axis_name="s",
                                        num_cores=1),
            out_type=jax.ShapeDtypeStruct((_B * _CHUNKS_PER_B * 9 * _CHUNK,),
                                          jnp.float32),
            scratch_types=[
                pltpu.VMEM((3 * _F,), jnp.float32),
                pltpu.VMEM((_NN * _CHUNK,), jnp.int32),
                pltpu.VMEM((9 * _CHUNK,), jnp.float32),
                pltpu.SemaphoreType.DMA,
            ],
            compiler_params=pltpu.CompilerParams(needs_layout_passes=False),
        ))
    return _sc_gather_cache[0](normals_flat, idx_flat)


def _tc_body(normals_ref, gathered_ref, wa_ref, wb_ref, g_ref, bb_ref, out_ref):
    alpha = wa_ref[...]
    beta = wb_ref[...]
    sa = jnp.sin(alpha)
    wx = sa * jnp.cos(beta)
    wy = sa * jnp.sin(beta)
    wz = jnp.cos(alpha)
    acc = jnp.zeros((_B, _K, _F), jnp.float32)
    for p in range(_NN + 1):
        if p == 0:
            px = normals_ref[:, 0, :]
            py = normals_ref[:, 1, :]
            pz = normals_ref[:, 2, :]
        else:
            px = gathered_ref[:, 0, p - 1, :]
            py = gathered_ref[:, 1, p - 1, :]
            pz = gathered_ref[:, 2, p - 1, :]
        pxb = px[:, None, :]
        pyb = py[:, None, :]
        pzb = pz[:, None, :]
        for m in range(4):
            dx = pxb - wx[m][None, :, None]
            dy = pyb - wy[m][None, :, None]
            dz = pzb - wz[m][None, :, None]
            d2 = dx * dx + dy * dy + dz * dz
            acc = acc + jnp.exp(d2 * _NEG_INV)
    feat = acc * (1.0 / ((_NN + 1) * 4))
    n = float(_B * _F)
    mu = jnp.sum(jnp.sum(feat, axis=2, keepdims=True), axis=0, keepdims=True) * (1.0 / n)
    d = feat - mu
    var = jnp.sum(jnp.sum(d * d, axis=2, keepdims=True), axis=0, keepdims=True) * (1.0 / n)
    inv = lax.rsqrt(var + 1e-5)
    gamma = g_ref[...][0][None, :, None]
    bshift = bb_ref[...][0][None, :, None]
    out_ref[...] = jnp.maximum(d * inv * gamma + bshift, 0.0)


def _tc_compute(normals, gathered, wa, wb, gamma, bbeta):
    return pl.pallas_call(
        _tc_body,
        out_shape=jax.ShapeDtypeStruct((_B, _K, _F), jnp.float32),
    )(normals, gathered, wa, wb, gamma, bbeta)


def kernel(normals, neighbor_index, weight_alpha, weight_beta, bn_gamma, bn_beta):
    idx_pre = neighbor_index.astype(jnp.int32).reshape(
        _B, _CHUNKS_PER_B, _CHUNK, _NN)
    idx_flat = jnp.transpose(idx_pre, (0, 1, 3, 2)).reshape(-1)
    gathered = _sc_gather(normals.reshape(-1), idx_flat)
    gathered = jnp.transpose(
        gathered.reshape(_B, _CHUNKS_PER_B, 9, _CHUNK),
        (0, 2, 1, 3)).reshape(_B, 3, _NN, _F)
    wa = jnp.transpose(weight_alpha[0])
    wb = jnp.transpose(weight_beta[0])
    return _tc_compute(normals, gathered, wa, wb,
                       bn_gamma.reshape(1, _K), bn_beta.reshape(1, _K))


# speedup vs baseline: 1.0509x; 1.0509x over previous
import jax
import jax.numpy as jnp
import numpy as np
from jax import lax
from jax.experimental import pallas as pl
from jax.experimental.pallas import tpu as pltpu
from jax.experimental.pallas import tpu_sc as plsc

_B, _K, _F, _NN = 8, 64, 1024, 3
_SIGMA = 0.2
_NEG_INV = -1.0 / (2.0 * _SIGMA * _SIGMA)
_NWORKERS = 16
_CHUNKS_PER_B = _NWORKERS // _B
_CHUNK = _F // _CHUNKS_PER_B


def _sc_gather_body(normals_hbm, idx_hbm, out_hbm, tbl_v, idx_v, out_v, sem):
    wid = lax.axis_index("s")
    b = wid // _CHUNKS_PER_B
    q = wid % _CHUNKS_PER_B
    cps = [
        pltpu.async_copy(normals_hbm.at[pl.ds(b * 3 * _F, 3 * _F)], tbl_v, sem),
        pltpu.async_copy(
            idx_hbm.at[pl.ds((b * _CHUNKS_PER_B + q) * _NN * _CHUNK,
                             _NN * _CHUNK)], idx_v, sem),
    ]
    for cp in cps:
        cp.wait()
    for c in range(3):
        for j in range(_NN):
            for i in range(_CHUNK // 16):
                iv = idx_v[pl.ds(j * _CHUNK + i * 16, 16)]
                out_v[pl.ds((c * _NN + j) * _CHUNK + i * 16, 16)] = (
                    plsc.load_gather(tbl_v, [iv + c * _F]))
    pltpu.sync_copy(
        out_v,
        out_hbm.at[pl.ds((b * _CHUNKS_PER_B + q) * 9 * _CHUNK, 9 * _CHUNK)])


_sc_gather_cache = []


def _sc_gather(normals_flat, idx_flat):
    if not _sc_gather_cache:
        _sc_gather_cache.append(pl.kernel(
            _sc_gather_body,
            mesh=plsc.VectorSubcoreMesh(core_axis_name="c", subcore_axis_name="s",
                                        num_cores=1),
            out_type=jax.ShapeDtypeStruct((_B * _CHUNKS_PER_B * 9 * _CHUNK,),
                                          jnp.float32),
            scratch_types=[
                pltpu.VMEM((3 * _F,), jnp.float32),
                pltpu.VMEM((_NN * _CHUNK,), jnp.int32),
                pltpu.VMEM((9 * _CHUNK,), jnp.float32),
                pltpu.SemaphoreType.DMA,
            ],
            compiler_params=pltpu.CompilerParams(needs_layout_passes=False),
        ))
    return _sc_gather_cache[0](normals_flat, idx_flat)


_INV_S2 = 1.0 / (_SIGMA * _SIGMA)


def _weights(wa_ref, wb_ref):
    alpha = wa_ref[...]
    beta = wb_ref[...]
    sa = jnp.sin(alpha)
    wx = sa * jnp.cos(beta)
    wy = sa * jnp.sin(beta)
    wz = jnp.cos(alpha)
    wxs = wx * _INV_S2
    wys = wy * _INV_S2
    wzs = wz * _INV_S2
    sw = (wx * wx + wy * wy + wz * wz) * _NEG_INV
    return wxs, wys, wzs, sw


def _accum_point(acc, px, py, pz, wxs, wys, wzs, sw):
    sp = (px * px + py * py + pz * pz) * _NEG_INV
    pxb = px[:, None, :]
    pyb = py[:, None, :]
    pzb = pz[:, None, :]
    spb = sp[:, None, :]
    for m in range(4):
        e = spb + sw[m][None, :, None]
        t = pxb * wxs[m][None, :, None] + pyb * wys[m][None, :, None] \
            + pzb * wzs[m][None, :, None] + e
        acc = acc + jnp.exp(t)
    return acc


def _tc_center_body(normals_ref, wa_ref, wb_ref, acc_ref):
    wxs, wys, wzs, sw = _weights(wa_ref, wb_ref)
    acc = jnp.zeros((_B, _K, _F), jnp.float32)
    acc = _accum_point(acc, normals_ref[:, 0, :], normals_ref[:, 1, :],
                       normals_ref[:, 2, :], wxs, wys, wzs, sw)
    acc_ref[...] = acc


def _tc_main_body(acc0_ref, gathered_ref, wa_ref, wb_ref, g_ref, bb_ref,
                  out_ref):
    wxs, wys, wzs, sw = _weights(wa_ref, wb_ref)
    acc = acc0_ref[...]
    for p in range(_NN):
        acc = _accum_point(acc, gathered_ref[:, 0, p, :],
                           gathered_ref[:, 1, p, :], gathered_ref[:, 2, p, :],
                           wxs, wys, wzs, sw)
    n = float(_B * _F)
    s1 = jnp.sum(jnp.sum(acc, axis=2, keepdims=True), axis=0, keepdims=True)
    s2 = jnp.sum(jnp.sum(acc * acc, axis=2, keepdims=True), axis=0,
                 keepdims=True)
    mu = s1 * (1.0 / n)
    var = s2 * (1.0 / n) - mu * mu
    inv16 = lax.rsqrt(var + 1e-5 * 256.0)
    gamma = g_ref[...][0][None, :, None]
    bshift = bb_ref[...][0][None, :, None]
    scale = inv16 * gamma
    shift = bshift - mu * scale
    out_ref[...] = jnp.maximum(acc * scale + shift, 0.0)


def kernel(normals, neighbor_index, weight_alpha, weight_beta, bn_gamma, bn_beta):
    idx_pre = neighbor_index.astype(jnp.int32).reshape(
        _B, _CHUNKS_PER_B, _CHUNK, _NN)
    idx_flat = jnp.transpose(idx_pre, (0, 1, 3, 2)).reshape(-1)
    gathered = _sc_gather(normals.reshape(-1), idx_flat)
    gathered = jnp.transpose(
        gathered.reshape(_B, _CHUNKS_PER_B, 9, _CHUNK),
        (0, 2, 1, 3)).reshape(_B, 3, _NN, _F)
    wa = jnp.transpose(weight_alpha[0])
    wb = jnp.transpose(weight_beta[0])
    acc0 = pl.pallas_call(
        _tc_center_body,
        out_shape=jax.ShapeDtypeStruct((_B, _K, _F), jnp.float32),
    )(normals, wa, wb)
    return pl.pallas_call(
        _tc_main_body,
        out_shape=jax.ShapeDtypeStruct((_B, _K, _F), jnp.float32),
    )(acc0, gathered, wa, wb, bn_gamma.reshape(1, _K), bn_beta.reshape(1, _K))


# speedup vs baseline: 1.2728x; 1.2112x over previous
import jax
import jax.numpy as jnp
import numpy as np
from jax import lax
from jax.experimental import pallas as pl
from jax.experimental.pallas import tpu as pltpu
from jax.experimental.pallas import tpu_sc as plsc

_B, _K, _F, _NN = 8, 64, 1024, 3
_SIGMA = 0.2
_NEG_INV = -1.0 / (2.0 * _SIGMA * _SIGMA)
_NWORKERS = 16
_CHUNKS_PER_B = _NWORKERS // _B
_CHUNK = _F // _CHUNKS_PER_B


def _sc_gather_body(normals_hbm, idx_hbm, out_hbm, tbl_v, idx_v, out_v, sem):
    wid = lax.axis_index("s")
    b = wid // _CHUNKS_PER_B
    q = wid % _CHUNKS_PER_B
    cps = [
        pltpu.async_copy(normals_hbm.at[pl.ds(b * 3 * _F, 3 * _F)], tbl_v, sem),
        pltpu.async_copy(
            idx_hbm.at[pl.ds((b * _CHUNKS_PER_B + q) * _NN * _CHUNK,
                             _NN * _CHUNK)], idx_v, sem),
    ]
    for cp in cps:
        cp.wait()
    for c in range(3):
        for j in range(_NN):
            for i in range(_CHUNK // 16):
                iv = idx_v[pl.ds(j * _CHUNK + i * 16, 16)]
                out_v[pl.ds((c * _NN + j) * _CHUNK + i * 16, 16)] = (
                    plsc.load_gather(tbl_v, [iv + c * _F]))
    pltpu.sync_copy(
        out_v,
        out_hbm.at[pl.ds((b * _CHUNKS_PER_B + q) * 9 * _CHUNK, 9 * _CHUNK)])


_sc_gather_cache = []


def _sc_gather(normals_flat, idx_flat):
    if not _sc_gather_cache:
        _sc_gather_cache.append(pl.kernel(
            _sc_gather_body,
            mesh=plsc.VectorSubcoreMesh(core_axis_name="c", subcore_axis_name="s",
                                        num_cores=1),
            out_type=jax.ShapeDtypeStruct((_B * _CHUNKS_PER_B * 9 * _CHUNK,),
                                          jnp.float32),
            scratch_types=[
                pltpu.VMEM((3 * _F,), jnp.float32),
                pltpu.VMEM((_NN * _CHUNK,), jnp.int32),
                pltpu.VMEM((9 * _CHUNK,), jnp.float32),
                pltpu.SemaphoreType.DMA,
            ],
            compiler_params=pltpu.CompilerParams(needs_layout_passes=False),
        ))
    return _sc_gather_cache[0](normals_flat, idx_flat)


_INV_S2 = 1.0 / (_SIGMA * _SIGMA)


def _weights(wa_ref, wb_ref):
    alpha = wa_ref[...]
    beta = wb_ref[...]
    sa = jnp.sin(alpha)
    wx = sa * jnp.cos(beta)
    wy = sa * jnp.sin(beta)
    wz = jnp.cos(alpha)
    sw = (wx * wx + wy * wy + wz * wz) * _NEG_INV
    ones = jnp.ones((4, _K), jnp.float32)
    wmat = jnp.stack([wx * _INV_S2, wy * _INV_S2, wz * _INV_S2, ones, sw],
                     axis=2)
    return wmat


def _point_mat(px, py, pz):
    sp = (px * px + py * py + pz * pz) * _NEG_INV
    ones = jnp.ones((_B, _F), jnp.float32)
    return jnp.stack([px, py, pz, sp, ones], axis=1)


def _accum_point(acc, pmat, wmat):
    for m in range(4):
        wb = jnp.broadcast_to(wmat[m][None], (_B, _K, 5))
        t = lax.dot_general(wb, pmat, (((2,), (1,)), ((0,), (0,))),
                            preferred_element_type=jnp.float32)
        acc = acc + jnp.exp(t)
    return acc


def _tc_center_body(normals_ref, wa_ref, wb_ref, acc_ref):
    wmat = _weights(wa_ref, wb_ref)
    pmat = _point_mat(normals_ref[:, 0, :], normals_ref[:, 1, :],
                      normals_ref[:, 2, :])
    acc = jnp.zeros((_B, _K, _F), jnp.float32)
    acc_ref[...] = _accum_point(acc, pmat, wmat)


def _tc_main_body(acc0_ref, gathered_ref, wa_ref, wb_ref, g_ref, bb_ref,
                  out_ref):
    wmat = _weights(wa_ref, wb_ref)
    acc = acc0_ref[...]
    for p in range(_NN):
        pmat = _point_mat(gathered_ref[:, 0, p, :], gathered_ref[:, 1, p, :],
                          gathered_ref[:, 2, p, :])
        acc = _accum_point(acc, pmat, wmat)
    n = float(_B * _F)
    s1 = jnp.sum(jnp.sum(acc, axis=2, keepdims=True), axis=0, keepdims=True)
    s2 = jnp.sum(jnp.sum(acc * acc, axis=2, keepdims=True), axis=0,
                 keepdims=True)
    mu = s1 * (1.0 / n)
    var = s2 * (1.0 / n) - mu * mu
    inv16 = lax.rsqrt(var + 1e-5 * 256.0)
    gamma = g_ref[...][0][None, :, None]
    bshift = bb_ref[...][0][None, :, None]
    scale = inv16 * gamma
    shift = bshift - mu * scale
    out_ref[...] = jnp.maximum(acc * scale + shift, 0.0)


def kernel(normals, neighbor_index, weight_alpha, weight_beta, bn_gamma, bn_beta):
    idx_pre = neighbor_index.astype(jnp.int32).reshape(
        _B, _CHUNKS_PER_B, _CHUNK, _NN)
    idx_flat = jnp.transpose(idx_pre, (0, 1, 3, 2)).reshape(-1)
    gathered = _sc_gather(normals.reshape(-1), idx_flat)
    gathered = jnp.transpose(
        gathered.reshape(_B, _CHUNKS_PER_B, 9, _CHUNK),
        (0, 2, 1, 3)).reshape(_B, 3, _NN, _F)
    wa = jnp.transpose(weight_alpha[0])
    wb = jnp.transpose(weight_beta[0])
    acc0 = pl.pallas_call(
        _tc_center_body,
        out_shape=jax.ShapeDtypeStruct((_B, _K, _F), jnp.float32),
    )(normals, wa, wb)
    return pl.pallas_call(
        _tc_main_body,
        out_shape=jax.ShapeDtypeStruct((_B, _K, _F), jnp.float32),
    )(acc0, gathered, wa, wb, bn_gamma.reshape(1, _K), bn_beta.reshape(1, _K))
